# baseline (speedup 1.0000x reference)
import functools

import jax
import jax.numpy as jnp
from jax import lax
from jax.experimental import pallas as pl
from jax.experimental.pallas import tpu as pltpu
from jax.experimental.pallas import tpu_sc as plsc

B, L, D = 4, 2048, 1024
NW = 32
CHUNK = (B * L) // NW

_SC_MESH = plsc.VectorSubcoreMesh(core_axis_name="c", subcore_axis_name="s")


def _select_body(_, bank_ref, o_ref):
    o_ref[...] = bank_ref[0]


def _tc_select(mask_bank, idx):
    return pl.pallas_call(
        _select_body,
        grid_spec=pltpu.PrefetchScalarGridSpec(
            num_scalar_prefetch=1,
            grid=(1,),
            in_specs=[
                pl.BlockSpec((1, B, L), lambda g, s: (s[0], 0, 0)),
            ],
            out_specs=pl.BlockSpec((B, L), lambda g, s: (0, 0)),
        ),
        out_shape=jax.ShapeDtypeStruct((B, L), jnp.float32),
    )(idx, mask_bank)


@functools.partial(
    pl.kernel,
    out_type=jax.ShapeDtypeStruct((B, L), jnp.float32),
    mesh=_SC_MESH,
    scratch_types=[
        pltpu.VMEM((272,), jnp.int32),
        pltpu.VMEM((2, 128), jnp.int32),
        pltpu.VMEM((256,), jnp.float32),
        pltpu.SemaphoreType.DMA,
    ],
    compiler_params=pltpu.CompilerParams(needs_layout_passes=False),
)
def _sc_mask_gather(row_hbm, packed_hbm, out_hbm, pk_v, idxv, vals, sem):
    w = lax.axis_index("s") * 2 + lax.axis_index("c")
    b = w // (L // CHUNK)
    r = w % (L // CHUNK)
    pltpu.sync_copy(packed_hbm.at[r], pk_v)
    lanes = lax.iota(jnp.int32, 16)
    rp_s = jnp.max(jnp.where(lanes == b, pk_v[pl.ds(256, 16)], 0))
    base = rp_s * L
    for c in range(2):
        for t in range(8):
            idxv[c, pl.ds(t * 16, 16)] = pk_v[pl.ds(c * 128 + t * 16, 16)] + base
    cps = [pltpu.async_copy(row_hbm.at[idxv.at[c]], vals.at[pl.ds(c * 128, 128)],
                            sem)
           for c in range(2)]
    for cp in cps:
        cp.wait()
    pltpu.sync_copy(vals, out_hbm.at[b, pl.ds(r * CHUNK, CHUNK)])


def _mul_body(x_ref, m_ref, o_ref):
    o_ref[...] = x_ref[...] * m_ref[0][:, :, None]


def _tc_mul(x, mask):
    return pl.pallas_call(
        _mul_body,
        out_shape=jax.ShapeDtypeStruct((B, L, D), jnp.float32),
        grid=(B,),
        in_specs=[
            pl.BlockSpec((1, L, D), lambda b: (b, 0, 0)),
            pl.BlockSpec((1, 1, L), lambda b: (b, 0, 0)),
        ],
        out_specs=pl.BlockSpec((1, L, D), lambda b: (b, 0, 0)),
        compiler_params=pltpu.CompilerParams(
            dimension_semantics=("arbitrary",),
            vmem_limit_bytes=110 * 1024 * 1024),
    )(x, mask)


def kernel(x, mask_bank, i, row_perm, col_perm):
    i32 = jnp.int32
    row = lax.dynamic_slice_in_dim(mask_bank, jnp.asarray(i, i32), 1,
                                   axis=0).reshape(-1)
    rp_pad = jnp.zeros((L // CHUNK, 16), i32).at[:, :B].set(
        row_perm.astype(i32)[None, :])
    packed = jnp.concatenate(
        [col_perm.astype(i32).reshape(L // CHUNK, CHUNK), rp_pad], axis=1)
    mask = _sc_mask_gather(row, packed).reshape(B, 1, L)
    return _tc_mul(x, mask)


# speedup vs baseline: 1.0618x; 1.0618x over previous
import functools

import jax
import jax.numpy as jnp
from jax import lax
from jax.experimental import pallas as pl
from jax.experimental.pallas import tpu as pltpu
from jax.experimental.pallas import tpu_sc as plsc

B, L, D = 4, 2048, 1024
NW = 32
CHUNK = (B * L) // NW

_SC_MESH = plsc.VectorSubcoreMesh(core_axis_name="c", subcore_axis_name="s")


def _select_body(_, bank_ref, o_ref):
    o_ref[...] = bank_ref[0]


def _tc_select(mask_bank, idx):
    return pl.pallas_call(
        _select_body,
        grid_spec=pltpu.PrefetchScalarGridSpec(
            num_scalar_prefetch=1,
            grid=(1,),
            in_specs=[
                pl.BlockSpec((1, B, L), lambda g, s: (s[0], 0, 0)),
            ],
            out_specs=pl.BlockSpec((B, L), lambda g, s: (0, 0)),
        ),
        out_shape=jax.ShapeDtypeStruct((B, L), jnp.float32),
    )(idx, mask_bank)


@functools.partial(
    pl.kernel,
    out_type=jax.ShapeDtypeStruct((B, L), jnp.float32),
    mesh=_SC_MESH,
    scratch_types=[
        pltpu.VMEM((272,), jnp.int32),
        pltpu.VMEM((2, 128), jnp.int32),
        pltpu.VMEM((256,), jnp.float32),
        pltpu.SemaphoreType.DMA,
    ],
    compiler_params=pltpu.CompilerParams(needs_layout_passes=False),
)
def _sc_mask_gather(row_hbm, packed_hbm, out_hbm, pk_v, idxv, vals, sem):
    w = lax.axis_index("s") * 2 + lax.axis_index("c")
    b = w // (L // CHUNK)
    r = w % (L // CHUNK)
    pltpu.sync_copy(packed_hbm.at[r], pk_v)
    lanes = lax.iota(jnp.int32, 16)
    rp_s = jnp.max(jnp.where(lanes == b, pk_v[pl.ds(256, 16)], 0))
    base = rp_s * L
    for c in range(2):
        for t in range(8):
            idxv[c, pl.ds(t * 16, 16)] = pk_v[pl.ds(c * 128 + t * 16, 16)] + base
    cps = [pltpu.async_copy(row_hbm.at[idxv.at[c]], vals.at[pl.ds(c * 128, 128)],
                            sem)
           for c in range(2)]
    for cp in cps:
        cp.wait()
    pltpu.sync_copy(vals, out_hbm.at[b, pl.ds(r * CHUNK, CHUNK)])


def _mul_body(x_ref, m_ref, o_ref):
    o_ref[...] = x_ref[...] * m_ref[...][:, :, None]


_BLK = 896


def _tc_mul(x, mask):
    blk = _BLK
    return pl.pallas_call(
        _mul_body,
        out_shape=jax.ShapeDtypeStruct((B, L, D), jnp.float32),
        grid=(pl.cdiv(L, _BLK),),
        in_specs=[
            pl.BlockSpec((B, blk, D), lambda l: (0, l, 0)),
            pl.BlockSpec((B, blk), lambda l: (0, l)),
        ],
        out_specs=pl.BlockSpec((B, blk, D), lambda l: (0, l, 0)),
        compiler_params=pltpu.CompilerParams(
            dimension_semantics=("arbitrary",),
            vmem_limit_bytes=110 * 1024 * 1024),
    )(x, mask)


def kernel(x, mask_bank, i, row_perm, col_perm):
    i32 = jnp.int32
    row = lax.dynamic_slice_in_dim(mask_bank, jnp.asarray(i, i32), 1,
                                   axis=0).reshape(-1)
    rp_pad = jnp.zeros((L // CHUNK, 16), i32).at[:, :B].set(
        row_perm.astype(i32)[None, :])
    packed = jnp.concatenate(
        [col_perm.astype(i32).reshape(L // CHUNK, CHUNK), rp_pad], axis=1)
    mask = _sc_mask_gather(row, packed)
    return _tc_mul(x, mask)
